# Initial kernel scaffold; baseline (speedup 1.0000x reference)
import jax
import jax.numpy as jnp
from jax.experimental import pallas as pl


def kernel(X, y, batch, batch_size, n_classes, samples_set_per_batch):
    raise NotImplementedError("write your pallas kernel here")


# speedup vs baseline: 4.4054x; 4.4054x over previous
import functools

import jax
import jax.numpy as jnp
import numpy as np
from jax import lax
from jax.experimental import pallas as pl
from jax.experimental.pallas import tpu as pltpu
from jax.experimental.pallas import tpu_sc as plsc

TWO_PI_E = 2.0 * np.pi * np.e
EPS = 1e-6

B = 16384
D = 64
G = 8
C = 2
NSEG = G * C
NC = 2
NS = 16
NW = NC * NS
RPW = B // NW
L = 16
ACC_W = 2 * D + L


def _sc_body(x_hbm, y_hbm, b_hbm, out_hbm, x_v, y_v, b_v, gid_v, acc_v):
    cid = lax.axis_index("c")
    sid = lax.axis_index("s")
    wid = cid * NS + sid
    base = wid * RPW

    pltpu.sync_copy(x_hbm.at[pl.ds(base, RPW)], x_v)
    pltpu.sync_copy(y_hbm.at[pl.ds(base, RPW)], y_v)
    pltpu.sync_copy(b_hbm.at[pl.ds(base, RPW)], b_v)

    zeros = jnp.zeros((L,), jnp.float32)
    for r in range(NSEG):
        for k in range(ACC_W // L):
            acc_v[r, pl.ds(k * L, L)] = zeros

    two = jnp.full((L,), C, jnp.int32)
    for i in range(RPW // L):
        sl = pl.ds(i * L, L)
        gid_v[sl] = b_v[sl] * two + y_v[sl]

    ones = jnp.full((L,), 1.0, jnp.float32)

    def chunk_body(i, carry):
        gvec = gid_v[pl.ds(i * L, L)]
        for lane in range(L):
            g = gvec[lane]
            r = i * L + lane
            for k in range(D // L):
                xk = x_v[r, pl.ds(k * L, L)]
                plsc.addupdate(acc_v.at[g, pl.ds(k * L, L)], xk)
                plsc.addupdate(acc_v.at[g, pl.ds(D + k * L, L)], xk * xk)
            plsc.addupdate(acc_v.at[g, pl.ds(2 * D, L)], ones)
        return carry

    lax.fori_loop(0, RPW // L, chunk_body, 0)

    pltpu.sync_copy(acc_v, out_hbm.at[pl.ds(wid * NSEG, NSEG)])


@functools.partial(
    pl.kernel,
    out_type=jax.ShapeDtypeStruct((NW * NSEG, ACC_W), jnp.float32),
    mesh=plsc.VectorSubcoreMesh(core_axis_name="c", subcore_axis_name="s"),
    scratch_types=[
        pltpu.VMEM((RPW, D), jnp.float32),
        pltpu.VMEM((RPW,), jnp.int32),
        pltpu.VMEM((RPW,), jnp.int32),
        pltpu.VMEM((RPW,), jnp.int32),
        pltpu.VMEM((NSEG, ACC_W), jnp.float32),
    ],
)
def _sc_partials(x_hbm, y_hbm, b_hbm, out_hbm, x_v, y_v, b_v, gid_v, acc_v):
    _sc_body(x_hbm, y_hbm, b_hbm, out_hbm, x_v, y_v, b_v, gid_v, acc_v)


def _fin_body(part_ref, out_ref):
    part = part_ref[...]

    ri = lax.broadcasted_iota(jnp.int32, (NSEG, NW * NSEG), 0)
    rj = lax.broadcasted_iota(jnp.int32, (NSEG, NW * NSEG), 1)
    R = jnp.where(rj % NSEG == ri, 1.0, 0.0).astype(jnp.float32)
    acc = jnp.dot(R, part, preferred_element_type=jnp.float32, precision=lax.Precision.HIGHEST)

    S_gc = acc[:, 0:D]
    Q_gc = acc[:, D:2 * D]
    N_gc = acc[:, 2 * D:2 * D + 1]

    Nc = jnp.maximum(N_gc, 1.0)
    mean_gc = S_gc / Nc
    var_gc = jnp.maximum(Q_gc / Nc - mean_gc * mean_gc, EPS)
    H_gc = 0.5 * jnp.sum(jnp.log(TWO_PI_E * var_gc), axis=1, keepdims=True)

    pi = lax.broadcasted_iota(jnp.int32, (G, NSEG), 0)
    pj = lax.broadcasted_iota(jnp.int32, (G, NSEG), 1)
    P = jnp.where(pj // C == pi, 1.0, 0.0).astype(jnp.float32)

    N_g = jnp.dot(P, N_gc, preferred_element_type=jnp.float32, precision=lax.Precision.HIGHEST)
    S_g = jnp.dot(P, S_gc, preferred_element_type=jnp.float32, precision=lax.Precision.HIGHEST)
    Q_g = jnp.dot(P, Q_gc, preferred_element_type=jnp.float32, precision=lax.Precision.HIGHEST)

    Ng = jnp.maximum(N_g, 1.0)
    mean_g = S_g / Ng
    var_g = jnp.maximum(Q_g / Ng - mean_g * mean_g, EPS)
    H_marg = 0.5 * jnp.sum(jnp.log(TWO_PI_E * var_g), axis=1, keepdims=True)

    Ng_rows = jnp.dot(P.T, Ng, preferred_element_type=jnp.float32, precision=lax.Precision.HIGHEST)
    p_gc = N_gc / Ng_rows
    H_cond = jnp.dot(P, p_gc * H_gc, preferred_element_type=jnp.float32, precision=lax.Precision.HIGHEST)

    mi = H_marg - H_cond
    out_ref[...] = jnp.broadcast_to(-jnp.sum(mi) * (1.0 / G), (1, 1))


def kernel(X, y, batch, batch_size, n_classes, samples_set_per_batch):
    X = X.astype(jnp.float32)
    y = y.astype(jnp.int32)
    batch = batch.astype(jnp.int32)
    part = _sc_partials(X, y, batch)
    out = pl.pallas_call(
        _fin_body,
        out_shape=jax.ShapeDtypeStruct((1, 1), jnp.float32),
    )(part)
    return out[0, 0]
